# baseline (speedup 1.0000x reference)
import functools

import jax
import jax.numpy as jnp
from jax import lax
from jax.experimental import pallas as pl
from jax.experimental.pallas import tpu as pltpu
from jax.experimental.pallas import tpu_sc as plsc

N = 10000
E = 320000
H = 128
K = 16
EPS = 0.2

NPAD = N + 48
CH = 64
NCHUNK = E // CH
NCHUNK_PAD = 5120
EXTRA = NCHUNK_PAD * CH - E
GRP = 8

_BLK = 1000


def _enc_body(x_ref, w1t_ref, b1_ref, alw_ref, arw_ref, ab_ref,
              h_ref, alr_ref):
    x = x_ref[...]
    z = jnp.dot(x, w1t_ref[...], preferred_element_type=jnp.float32)
    z = z + b1_ref[...]
    h = jnp.where(z > 0, z, jnp.exp(z) - 1.0)
    h_ref[...] = h
    al = jnp.sum(h * alw_ref[...], axis=1, keepdims=True) + ab_ref[0, 0]
    ar = jnp.sum(h * arw_ref[...], axis=1, keepdims=True) + ab_ref[0, 1]
    alr_ref[...] = jnp.concatenate([al, ar], axis=1)


def _encode(X, W1T, b1r, alw, arw, ab):
    grid = N // _BLK
    return pl.pallas_call(
        _enc_body,
        grid=(grid,),
        in_specs=[
            pl.BlockSpec((_BLK, H), lambda i: (i, 0)),
            pl.BlockSpec((H, H), lambda i: (0, 0)),
            pl.BlockSpec((1, H), lambda i: (0, 0)),
            pl.BlockSpec((1, H), lambda i: (0, 0)),
            pl.BlockSpec((1, H), lambda i: (0, 0)),
            pl.BlockSpec(memory_space=pltpu.SMEM),
        ],
        out_specs=[
            pl.BlockSpec((_BLK, H), lambda i: (i, 0)),
            pl.BlockSpec((_BLK, 2), lambda i: (i, 0)),
        ],
        out_shape=[
            jax.ShapeDtypeStruct((N, H), jnp.float32),
            jax.ShapeDtypeStruct((N, 2), jnp.float32),
        ],
    )(X, W1T, b1r, alw, arw, ab)


def _rsqrt16(x):
    i = lax.bitcast_convert_type(x, jnp.int32)
    i = jnp.int32(0x5F3759DF) - lax.shift_right_logical(i, 1)
    y = lax.bitcast_convert_type(i, jnp.float32)
    for _ in range(3):
        y = y * (1.5 - 0.5 * x * y * y)
    return y


def _gconv_body(row2, col2, h_hbm, al_hbm, ar_hbm, out_hbm,
                agg_sh, deg_sh, al_v, ar_v, dinv_v, row8, col8,
                ones_b, rows_a, rows_b, coef_b, dstage,
                gsa, gsb, ssa, ssb):
    c = lax.axis_index("c")
    s = lax.axis_index("s")
    wid = s * 2 + c

    zero = jnp.zeros((16,), jnp.float32)
    one = jnp.ones((16,), jnp.float32)

    @plsc.parallel_loop(0, CH, unroll=8)
    def _init_body(i):
        for f in range(8):
            rows_a[i, pl.ds(f * 16, 16)] = zero
    for i in range(4):
        ones_b[pl.ds(i * 16, 16)] = one
    def _zstage_body(i, carry):
        dstage[pl.ds(i * 16, 16)] = zero
        return carry
    lax.fori_loop(0, 16, _zstage_body, None)

    pltpu.async_copy(al_hbm, al_v.at[pl.ds(0, N)], gsb)
    pltpu.async_copy(ar_hbm, ar_v.at[pl.ds(0, N)], gsb)
    for k in range(10):
        idx = k * 16 + s
        @pl.when(idx < NPAD // CH)
        def _():
            pltpu.async_copy(rows_a, agg_sh.at[pl.ds(idx * CH, CH)], ssa)
    for k in range(3):
        idx = k * 16 + s
        @pl.when(idx < 39)
        def _():
            pltpu.async_copy(dstage, deg_sh.at[pl.ds(idx * 256, 256)], ssb)
        @pl.when(idx == 39)
        def _():
            pltpu.async_copy(dstage.at[pl.ds(0, 64)],
                             deg_sh.at[pl.ds(9984, 64)], ssb)
    for t in range(3):
        al_v[pl.ds(N + t * 16, 16)] = zero
        ar_v[pl.ds(N + t * 16, 16)] = zero
    for k in range(10):
        idx = k * 16 + s
        @pl.when(idx < NPAD // CH)
        def _():
            pltpu.make_async_copy(rows_a, agg_sh.at[pl.ds(idx * CH, CH)],
                                  ssa).wait()
    for k in range(3):
        idx = k * 16 + s
        @pl.when(idx < 39)
        def _():
            pltpu.make_async_copy(dstage, deg_sh.at[pl.ds(idx * 256, 256)],
                                  ssb).wait()
        @pl.when(idx == 39)
        def _():
            pltpu.make_async_copy(dstage.at[pl.ds(0, 64)],
                                  deg_sh.at[pl.ds(9984, 64)], ssb).wait()
    pltpu.make_async_copy(al_hbm, al_v.at[pl.ds(0, N)], gsb).wait()
    pltpu.make_async_copy(ar_hbm, ar_v.at[pl.ds(0, N)], gsb).wait()

    plsc.subcore_barrier()

    def _deg_grp(i, carry):
        @pl.when(i > 0)
        def _():
            for _j in range(8):
                pltpu.make_async_copy(ones_b, deg_sh.at[col8.at[1]],
                                      gsa).wait()
        pltpu.sync_copy(col2.at[pl.ds(s * 320 + i * 16, 8)], col8)
        for j in range(8):
            pltpu.async_copy(ones_b, deg_sh.at[col8.at[j]], gsa, add=True)
        @pl.when(i > 0)
        def _():
            for _j in range(8):
                pltpu.make_async_copy(ones_b, deg_sh.at[row8.at[1]],
                                      gsb).wait()
        pltpu.sync_copy(col2.at[pl.ds(s * 320 + i * 16 + 8, 8)], row8)
        for j in range(8):
            pltpu.async_copy(ones_b, deg_sh.at[row8.at[j]], gsb, add=True)
        return carry
    lax.fori_loop(0, 20, _deg_grp, None)
    for _j in range(8):
        pltpu.make_async_copy(ones_b, deg_sh.at[col8.at[1]], gsa).wait()
        pltpu.make_async_copy(ones_b, deg_sh.at[row8.at[1]], gsb).wait()

    plsc.subcore_barrier()

    def _dinv_chunk(off, nrows):
        pltpu.sync_copy(deg_sh.at[pl.ds(off, nrows)],
                        dstage.at[pl.ds(0, nrows)])
        @plsc.parallel_loop(0, nrows // 16, unroll=4)
        def _dinv_body(i):
            d = dstage[pl.ds(i * 16, 16)]
            y = _rsqrt16(d)
            dstage[pl.ds(i * 16, 16)] = jnp.where(d > 0, y, 0.0)
        pltpu.sync_copy(dstage.at[pl.ds(0, nrows)],
                        deg_sh.at[pl.ds(off, nrows)])
    _dinv_chunk(s * 640, 256)
    @pl.when(s < 15)
    def _():
        _dinv_chunk(s * 640 + 256, 256)
        _dinv_chunk(s * 640 + 512, 128)
    @pl.when(s == 15)
    def _():
        _dinv_chunk(9856, 128)
        _dinv_chunk(9984, 64)

    plsc.subcore_barrier()
    pltpu.sync_copy(deg_sh, dinv_v)

    bufs = (rows_a, rows_b)
    gsems = (gsa, gsb)
    ssems = (ssa, ssb)

    def _chunk_compute(X, j):
        @plsc.parallel_loop(0, CH // 16, unroll=4)
        def _coef_body(i):
            rr = row8[j, pl.ds(i * 16, 16)]
            cc = col8[j, pl.ds(i * 16, 16)]
            a = plsc.load_gather(al_v, [rr]) + plsc.load_gather(ar_v, [cc])
            t = jnp.where(a >= 0, 1.0, -1.0) * (
                1.0 - 2.0 / (jnp.exp(2.0 * jnp.abs(a)) + 1.0))
            coef = (t * plsc.load_gather(dinv_v, [rr])
                    * plsc.load_gather(dinv_v, [cc]))
            coef_b[pl.ds(i * 16, 16)] = coef
        @plsc.parallel_loop(0, CH, unroll=16)
        def _scale_body(e):
            cv = plsc.load_gather(coef_b, [jnp.zeros((16,), jnp.int32) + e])
            for f in range(8):
                X[e, pl.ds(f * 16, 16)] = X[e, pl.ds(f * 16, 16)] * cv

    def _main_grp(gq, carry):
        gb = wid * 160 + gq * GRP
        pltpu.sync_copy(row2.at[pl.ds(gb, GRP)], row8)
        pltpu.sync_copy(col2.at[pl.ds(gb, GRP)], col8)
        @pl.when(gq > 0)
        def _():
            pltpu.make_async_copy(rows_a, agg_sh.at[col8.at[0]], ssa).wait()
        gd = {0: pltpu.async_copy(h_hbm.at[row8.at[0]], rows_a, gsa)}
        sd = {}
        for j in range(GRP):
            X = bufs[j % 2]
            if j + 1 < GRP:
                Y = bufs[(j + 1) % 2]
                if j == 0:
                    @pl.when(gq > 0)
                    def _():
                        pltpu.make_async_copy(
                            rows_b, agg_sh.at[col8.at[0]], ssb).wait()
                else:
                    sd[j - 1].wait()
                gd[j + 1] = pltpu.async_copy(h_hbm.at[row8.at[j + 1]], Y,
                                             gsems[(j + 1) % 2])
            gd[j].wait()
            _chunk_compute(X, j)
            sd[j] = pltpu.async_copy(X, agg_sh.at[col8.at[j]], ssems[j % 2],
                                     add=True)
        return carry
    lax.fori_loop(0, 160 // GRP, _main_grp, None)
    pltpu.make_async_copy(rows_a, agg_sh.at[col8.at[0]], ssa).wait()
    pltpu.make_async_copy(rows_b, agg_sh.at[col8.at[0]], ssb).wait()

    plsc.subcore_barrier()

    ob = s * 624
    pltpu.sync_copy(agg_sh.at[pl.ds(ob, 624)],
                    out_hbm.at[pl.ds(c * N + ob, 624)])
    @pl.when(s == 15)
    def _tail():
        pltpu.sync_copy(agg_sh.at[pl.ds(9984, 16)],
                        out_hbm.at[pl.ds(c * N + 9984, 16)])


def _gconv(row2, col2, h, al, ar):
    mesh = plsc.VectorSubcoreMesh(core_axis_name="c", subcore_axis_name="s",
                                  num_cores=2, num_subcores=16)
    f = pl.kernel(
        _gconv_body,
        out_type=jax.ShapeDtypeStruct((2 * N, H), jnp.float32),
        mesh=mesh,
        compiler_params=pltpu.CompilerParams(needs_layout_passes=False),
        scratch_types=[
            pltpu.VMEM_SHARED((NPAD, H), jnp.float32),
            pltpu.VMEM_SHARED((NPAD,), jnp.float32),
            pltpu.VMEM((NPAD,), jnp.float32),
            pltpu.VMEM((NPAD,), jnp.float32),
            pltpu.VMEM((NPAD,), jnp.float32),
            pltpu.VMEM((GRP, CH), jnp.int32),
            pltpu.VMEM((GRP, CH), jnp.int32),
            pltpu.VMEM((CH,), jnp.float32),
            pltpu.VMEM((CH, H), jnp.float32),
            pltpu.VMEM((CH, H), jnp.float32),
            pltpu.VMEM((CH,), jnp.float32),
            pltpu.VMEM((256,), jnp.float32),
            pltpu.SemaphoreType.DMA,
            pltpu.SemaphoreType.DMA,
            pltpu.SemaphoreType.DMA,
            pltpu.SemaphoreType.DMA,
        ],
    )
    return f(row2, col2, h, al, ar)


def _cls_body(a0_ref, a1_ref, h_ref, w2t_ref, b2_ref, o_ref):
    out = a0_ref[...] + a1_ref[...] + EPS * h_ref[...]
    logits = jnp.dot(out, w2t_ref[...], preferred_element_type=jnp.float32)
    logits = logits + b2_ref[...]
    m = jnp.max(logits, axis=1, keepdims=True)
    ex = jnp.exp(logits - m)
    o_ref[...] = ex / jnp.sum(ex, axis=1, keepdims=True)


def _classify(a0, a1, h, W2T, b2r):
    grid = N // _BLK
    return pl.pallas_call(
        _cls_body,
        grid=(grid,),
        in_specs=[
            pl.BlockSpec((_BLK, H), lambda i: (i, 0)),
            pl.BlockSpec((_BLK, H), lambda i: (i, 0)),
            pl.BlockSpec((_BLK, H), lambda i: (i, 0)),
            pl.BlockSpec((H, K), lambda i: (0, 0)),
            pl.BlockSpec((1, K), lambda i: (0, 0)),
        ],
        out_specs=pl.BlockSpec((_BLK, K), lambda i: (i, 0)),
        out_shape=jax.ShapeDtypeStruct((N, K), jnp.float32),
    )(a0, a1, h, W2T, b2r)


def kernel(X, ei_feat, batch, W1, b1, al_w, al_b, ar_w, ar_b, W2, b2):
    row = ei_feat[0]
    col = ei_feat[1]
    pad = jnp.arange(EXTRA, dtype=jnp.int32) % 48
    row2 = jnp.concatenate([row, pad]).reshape(NCHUNK_PAD, CH)
    col2 = jnp.concatenate([col, N + pad]).reshape(NCHUNK_PAD, CH)
    ab = jnp.stack([al_b[0], ar_b[0]]).reshape(1, 2)
    h, alr = _encode(X, W1.T, b1.reshape(1, H), al_w, ar_w, ab)
    aggf = _gconv(row2, col2, h, alr[:, 0], alr[:, 1])
    return _classify(aggf[:N], aggf[N:], h, W2.T, b2.reshape(1, K))


# speedup vs baseline: 1.0487x; 1.0487x over previous
import functools

import jax
import jax.numpy as jnp
from jax import lax
from jax.experimental import pallas as pl
from jax.experimental.pallas import tpu as pltpu
from jax.experimental.pallas import tpu_sc as plsc

N = 10000
E = 320000
H = 128
K = 16
EPS = 0.2

NPAD = N + 48
CH = 64
NCHUNK = E // CH
NCHUNK_PAD = 5120
EXTRA = NCHUNK_PAD * CH - E
GRP = 8

_BLK = 1000


def _enc_body(x_ref, w1t_ref, b1_ref, alw_ref, arw_ref, ab_ref,
              h_ref, alr_ref):
    x = x_ref[...]
    z = jnp.dot(x, w1t_ref[...], preferred_element_type=jnp.float32)
    z = z + b1_ref[...]
    h = jnp.where(z > 0, z, jnp.exp(z) - 1.0)
    h_ref[...] = h
    al = jnp.sum(h * alw_ref[...], axis=1, keepdims=True) + ab_ref[0, 0]
    ar = jnp.sum(h * arw_ref[...], axis=1, keepdims=True) + ab_ref[0, 1]
    alr_ref[...] = jnp.concatenate([al, ar], axis=1)


def _encode(X, W1T, b1r, alw, arw, ab):
    grid = N // _BLK
    return pl.pallas_call(
        _enc_body,
        grid=(grid,),
        in_specs=[
            pl.BlockSpec((_BLK, H), lambda i: (i, 0)),
            pl.BlockSpec((H, H), lambda i: (0, 0)),
            pl.BlockSpec((1, H), lambda i: (0, 0)),
            pl.BlockSpec((1, H), lambda i: (0, 0)),
            pl.BlockSpec((1, H), lambda i: (0, 0)),
            pl.BlockSpec(memory_space=pltpu.SMEM),
        ],
        out_specs=[
            pl.BlockSpec((_BLK, H), lambda i: (i, 0)),
            pl.BlockSpec((_BLK, 2), lambda i: (i, 0)),
        ],
        out_shape=[
            jax.ShapeDtypeStruct((N, H), jnp.float32),
            jax.ShapeDtypeStruct((N, 2), jnp.float32),
        ],
    )(X, W1T, b1r, alw, arw, ab)


def _rsqrt16(x):
    i = lax.bitcast_convert_type(x, jnp.int32)
    i = jnp.int32(0x5F3759DF) - lax.shift_right_logical(i, 1)
    y = lax.bitcast_convert_type(i, jnp.float32)
    for _ in range(3):
        y = y * (1.5 - 0.5 * x * y * y)
    return y


def _gconv_body(row2, col2, h_hbm, al_hbm, ar_hbm, out_hbm,
                agg_sh, deg_sh, al_v, ar_v, dinv_v, row8, col8,
                ones_b, rows_a, rows_b, coef_b, dstage,
                gsa, gsb, ssa, ssb):
    c = lax.axis_index("c")
    s = lax.axis_index("s")
    wid = s * 2 + c

    zero = jnp.zeros((16,), jnp.float32)
    one = jnp.ones((16,), jnp.float32)

    def _init_body(i, carry):
        for f in range(8):
            rows_a[i, pl.ds(f * 16, 16)] = zero
        return carry
    lax.fori_loop(0, CH, _init_body, None)
    for i in range(4):
        ones_b[pl.ds(i * 16, 16)] = one
    def _zstage_body(i, carry):
        dstage[pl.ds(i * 16, 16)] = zero
        return carry
    lax.fori_loop(0, 16, _zstage_body, None)

    pltpu.async_copy(al_hbm, al_v.at[pl.ds(0, N)], gsb)
    pltpu.async_copy(ar_hbm, ar_v.at[pl.ds(0, N)], gsb)
    for k in range(10):
        idx = k * 16 + s
        @pl.when(idx < NPAD // CH)
        def _():
            pltpu.async_copy(rows_a, agg_sh.at[pl.ds(idx * CH, CH)], ssa)
    for k in range(3):
        idx = k * 16 + s
        @pl.when(idx < 39)
        def _():
            pltpu.async_copy(dstage, deg_sh.at[pl.ds(idx * 256, 256)], ssb)
        @pl.when(idx == 39)
        def _():
            pltpu.async_copy(dstage.at[pl.ds(0, 64)],
                             deg_sh.at[pl.ds(9984, 64)], ssb)
    for t in range(3):
        al_v[pl.ds(N + t * 16, 16)] = zero
        ar_v[pl.ds(N + t * 16, 16)] = zero
    for k in range(10):
        idx = k * 16 + s
        @pl.when(idx < NPAD // CH)
        def _():
            pltpu.make_async_copy(rows_a, agg_sh.at[pl.ds(idx * CH, CH)],
                                  ssa).wait()
    for k in range(3):
        idx = k * 16 + s
        @pl.when(idx < 39)
        def _():
            pltpu.make_async_copy(dstage, deg_sh.at[pl.ds(idx * 256, 256)],
                                  ssb).wait()
        @pl.when(idx == 39)
        def _():
            pltpu.make_async_copy(dstage.at[pl.ds(0, 64)],
                                  deg_sh.at[pl.ds(9984, 64)], ssb).wait()
    pltpu.make_async_copy(al_hbm, al_v.at[pl.ds(0, N)], gsb).wait()
    pltpu.make_async_copy(ar_hbm, ar_v.at[pl.ds(0, N)], gsb).wait()

    plsc.subcore_barrier()

    def _deg_grp(i, carry):
        @pl.when(i > 0)
        def _():
            for _j in range(8):
                pltpu.make_async_copy(ones_b, deg_sh.at[col8.at[1]],
                                      gsa).wait()
        pltpu.sync_copy(col2.at[pl.ds(s * 320 + i * 16, 8)], col8)
        for j in range(8):
            pltpu.async_copy(ones_b, deg_sh.at[col8.at[j]], gsa, add=True)
        @pl.when(i > 0)
        def _():
            for _j in range(8):
                pltpu.make_async_copy(ones_b, deg_sh.at[row8.at[1]],
                                      gsb).wait()
        pltpu.sync_copy(col2.at[pl.ds(s * 320 + i * 16 + 8, 8)], row8)
        for j in range(8):
            pltpu.async_copy(ones_b, deg_sh.at[row8.at[j]], gsb, add=True)
        return carry
    lax.fori_loop(0, 20, _deg_grp, None)
    for _j in range(8):
        pltpu.make_async_copy(ones_b, deg_sh.at[col8.at[1]], gsa).wait()
        pltpu.make_async_copy(ones_b, deg_sh.at[row8.at[1]], gsb).wait()

    plsc.subcore_barrier()

    def _dinv_chunk(off, nrows):
        pltpu.sync_copy(deg_sh.at[pl.ds(off, nrows)],
                        dstage.at[pl.ds(0, nrows)])
        def _dinv_body(i, carry):
            d = dstage[pl.ds(i * 16, 16)]
            y = _rsqrt16(d)
            dstage[pl.ds(i * 16, 16)] = jnp.where(d > 0, y, 0.0)
            return carry
        lax.fori_loop(0, nrows // 16, _dinv_body, None)
        pltpu.sync_copy(dstage.at[pl.ds(0, nrows)],
                        deg_sh.at[pl.ds(off, nrows)])
    _dinv_chunk(s * 640, 256)
    @pl.when(s < 15)
    def _():
        _dinv_chunk(s * 640 + 256, 256)
        _dinv_chunk(s * 640 + 512, 128)
    @pl.when(s == 15)
    def _():
        _dinv_chunk(9856, 128)
        _dinv_chunk(9984, 64)

    plsc.subcore_barrier()
    pltpu.sync_copy(deg_sh, dinv_v)

    bufs = (rows_a, rows_b)
    gsems = (gsa, gsb)
    ssems = (ssa, ssb)

    def _chunk_compute(X, j):
        @plsc.parallel_loop(0, CH // 16, unroll=4)
        def _coef_body(i):
            rr = row8[j, pl.ds(i * 16, 16)]
            cc = col8[j, pl.ds(i * 16, 16)]
            a = plsc.load_gather(al_v, [rr]) + plsc.load_gather(ar_v, [cc])
            t = jnp.where(a >= 0, 1.0, -1.0) * (
                1.0 - 2.0 / (jnp.exp(2.0 * jnp.abs(a)) + 1.0))
            coef = (t * plsc.load_gather(dinv_v, [rr])
                    * plsc.load_gather(dinv_v, [cc]))
            coef_b[pl.ds(i * 16, 16)] = coef
        @plsc.parallel_loop(0, CH, unroll=8)
        def _scale_body(e):
            cv = plsc.load_gather(coef_b, [jnp.zeros((16,), jnp.int32) + e])
            for f in range(8):
                X[e, pl.ds(f * 16, 16)] = X[e, pl.ds(f * 16, 16)] * cv

    def _main_grp(gq, carry):
        gb = wid * 160 + gq * GRP
        pltpu.sync_copy(row2.at[pl.ds(gb, GRP)], row8)
        pltpu.sync_copy(col2.at[pl.ds(gb, GRP)], col8)
        @pl.when(gq > 0)
        def _():
            pltpu.make_async_copy(rows_a, agg_sh.at[col8.at[0]], ssa).wait()
        gd = {0: pltpu.async_copy(h_hbm.at[row8.at[0]], rows_a, gsa)}
        sd = {}
        for j in range(GRP):
            X = bufs[j % 2]
            if j + 1 < GRP:
                Y = bufs[(j + 1) % 2]
                if j == 0:
                    @pl.when(gq > 0)
                    def _():
                        pltpu.make_async_copy(
                            rows_b, agg_sh.at[col8.at[0]], ssb).wait()
                else:
                    sd[j - 1].wait()
                gd[j + 1] = pltpu.async_copy(h_hbm.at[row8.at[j + 1]], Y,
                                             gsems[(j + 1) % 2])
            gd[j].wait()
            _chunk_compute(X, j)
            sd[j] = pltpu.async_copy(X, agg_sh.at[col8.at[j]], ssems[j % 2],
                                     add=True)
        return carry
    lax.fori_loop(0, 160 // GRP, _main_grp, None)
    pltpu.make_async_copy(rows_a, agg_sh.at[col8.at[0]], ssa).wait()
    pltpu.make_async_copy(rows_b, agg_sh.at[col8.at[0]], ssb).wait()

    plsc.subcore_barrier()

    ob = s * 624
    pltpu.sync_copy(agg_sh.at[pl.ds(ob, 624)],
                    out_hbm.at[pl.ds(c * N + ob, 624)])
    @pl.when(s == 15)
    def _tail():
        pltpu.sync_copy(agg_sh.at[pl.ds(9984, 16)],
                        out_hbm.at[pl.ds(c * N + 9984, 16)])


def _gconv(row2, col2, h, al, ar):
    mesh = plsc.VectorSubcoreMesh(core_axis_name="c", subcore_axis_name="s",
                                  num_cores=2, num_subcores=16)
    f = pl.kernel(
        _gconv_body,
        out_type=jax.ShapeDtypeStruct((2 * N, H), jnp.float32),
        mesh=mesh,
        compiler_params=pltpu.CompilerParams(needs_layout_passes=False),
        scratch_types=[
            pltpu.VMEM_SHARED((NPAD, H), jnp.float32),
            pltpu.VMEM_SHARED((NPAD,), jnp.float32),
            pltpu.VMEM((NPAD,), jnp.float32),
            pltpu.VMEM((NPAD,), jnp.float32),
            pltpu.VMEM((NPAD,), jnp.float32),
            pltpu.VMEM((GRP, CH), jnp.int32),
            pltpu.VMEM((GRP, CH), jnp.int32),
            pltpu.VMEM((CH,), jnp.float32),
            pltpu.VMEM((CH, H), jnp.float32),
            pltpu.VMEM((CH, H), jnp.float32),
            pltpu.VMEM((CH,), jnp.float32),
            pltpu.VMEM((256,), jnp.float32),
            pltpu.SemaphoreType.DMA,
            pltpu.SemaphoreType.DMA,
            pltpu.SemaphoreType.DMA,
            pltpu.SemaphoreType.DMA,
        ],
    )
    return f(row2, col2, h, al, ar)


def _cls_body(a0_ref, a1_ref, h_ref, w2t_ref, b2_ref, o_ref):
    out = a0_ref[...] + a1_ref[...] + EPS * h_ref[...]
    logits = jnp.dot(out, w2t_ref[...], preferred_element_type=jnp.float32)
    logits = logits + b2_ref[...]
    m = jnp.max(logits, axis=1, keepdims=True)
    ex = jnp.exp(logits - m)
    o_ref[...] = ex / jnp.sum(ex, axis=1, keepdims=True)


def _classify(a0, a1, h, W2T, b2r):
    grid = N // _BLK
    return pl.pallas_call(
        _cls_body,
        grid=(grid,),
        in_specs=[
            pl.BlockSpec((_BLK, H), lambda i: (i, 0)),
            pl.BlockSpec((_BLK, H), lambda i: (i, 0)),
            pl.BlockSpec((_BLK, H), lambda i: (i, 0)),
            pl.BlockSpec((H, K), lambda i: (0, 0)),
            pl.BlockSpec((1, K), lambda i: (0, 0)),
        ],
        out_specs=pl.BlockSpec((_BLK, K), lambda i: (i, 0)),
        out_shape=jax.ShapeDtypeStruct((N, K), jnp.float32),
    )(a0, a1, h, W2T, b2r)


def kernel(X, ei_feat, batch, W1, b1, al_w, al_b, ar_w, ar_b, W2, b2):
    row = ei_feat[0]
    col = ei_feat[1]
    pad = jnp.arange(EXTRA, dtype=jnp.int32) % 48
    row2 = jnp.concatenate([row, pad]).reshape(NCHUNK_PAD, CH)
    col2 = jnp.concatenate([col, N + pad]).reshape(NCHUNK_PAD, CH)
    ab = jnp.stack([al_b[0], ar_b[0]]).reshape(1, 2)
    h, alr = _encode(X, W1.T, b1.reshape(1, H), al_w, ar_w, ab)
    aggf = _gconv(row2, col2, h, alr[:, 0], alr[:, 1])
    return _classify(aggf[:N], aggf[N:], h, W2.T, b2.reshape(1, K))
